# baseline (speedup 1.0000x reference)
import functools

import jax
import jax.numpy as jnp
from jax import lax
from jax.experimental import pallas as pl
from jax.experimental.pallas import tpu as pltpu
from jax.experimental.pallas import tpu_sc as plsc

N_CAT = 26
N_NUM = 13
VOCAB = 100001
EMBED = 32
HIDDEN = 64
BATCH = 16384

NW = 32
CB = BATCH // NW
SB = 64
NCH = CB // SB
ROWS = SB * N_CAT
KIDX = ROWS // 128


def _sc_gather_sum(tables_flat, idx):
    mesh = plsc.VectorSubcoreMesh(core_axis_name="c", subcore_axis_name="s")

    @functools.partial(
        pl.kernel,
        mesh=mesh,
        out_type=jax.ShapeDtypeStruct((BATCH, EMBED), jnp.float32),
        compiler_params=pltpu.CompilerParams(use_tc_tiling_on_sc=False),
        scratch_types=[
            pltpu.VMEM((KIDX, 128), jnp.int32),
            pltpu.VMEM((ROWS, EMBED), jnp.float32),
            pltpu.VMEM((SB, EMBED), jnp.float32),
            pltpu.SemaphoreType.DMA,
        ],
    )
    def k(tab_hbm, idx_hbm, out_hbm, idx_v, rows_v, out_v, sem):
        wid = lax.axis_index("s") * 2 + lax.axis_index("c")

        def chunk(c, carry):
            pltpu.sync_copy(idx_hbm.at[wid * NCH + c], idx_v)
            cps = [
                pltpu.async_copy(
                    tab_hbm.at[idx_v.at[j]],
                    rows_v.at[pl.ds(j * 128, 128)],
                    sem,
                )
                for j in range(KIDX)
            ]
            for cp in cps:
                cp.wait()

            def red(sb, carry2):
                base = sb * N_CAT
                for h in range(2):
                    acc = rows_v[base, pl.ds(h * 16, 16)]
                    for f in range(1, N_CAT):
                        acc = acc + rows_v[base + f, pl.ds(h * 16, 16)]
                    out_v[sb, pl.ds(h * 16, 16)] = acc
                return carry2

            lax.fori_loop(0, SB, red, 0)
            pltpu.sync_copy(out_v, out_hbm.at[pl.ds(wid * CB + c * SB, SB)])
            return carry

        lax.fori_loop(0, NCH, chunk, 0)

    return k(tables_flat, idx)


def _dense_tail(obj, nums, W1, W2):
    BM = 2048

    def body(obj_ref, num_ref, w1_ref, w2_ref, out_ref):
        x = obj_ref[...] + lax.dot_general(
            num_ref[...], w1_ref[...], (((1,), (1,)), ((), ())),
            preferred_element_type=jnp.float32)
        out_ref[...] = lax.dot_general(
            x, w2_ref[...], (((1,), (1,)), ((), ())),
            preferred_element_type=jnp.float32)

    return pl.pallas_call(
        body,
        grid=(BATCH // BM,),
        in_specs=[
            pl.BlockSpec((BM, EMBED), lambda i: (i, 0)),
            pl.BlockSpec((BM, N_NUM), lambda i: (i, 0)),
            pl.BlockSpec((EMBED, N_NUM), lambda i: (0, 0)),
            pl.BlockSpec((HIDDEN, EMBED), lambda i: (0, 0)),
        ],
        out_specs=pl.BlockSpec((BM, HIDDEN), lambda i: (i, 0)),
        out_shape=jax.ShapeDtypeStruct((BATCH, HIDDEN), jnp.float32),
    )(obj, nums, W1, W2)


def kernel(edge_feats, tables, W1, W2):
    cat = edge_feats[:, :N_CAT].astype(jnp.int32)
    flat = cat + (jnp.arange(N_CAT, dtype=jnp.int32) * VOCAB)[None, :]
    idx = flat.reshape(NW * NCH, KIDX, 128)
    obj = _sc_gather_sum(tables.reshape(N_CAT * VOCAB, EMBED), idx)
    return _dense_tail(obj, edge_feats[:, N_CAT:], W1, W2)


# speedup vs baseline: 16.8416x; 16.8416x over previous
import functools

import jax
import jax.numpy as jnp
from jax import lax
from jax.experimental import pallas as pl
from jax.experimental.pallas import tpu as pltpu
from jax.experimental.pallas import tpu_sc as plsc

N_CAT = 26
N_NUM = 13
VOCAB = 100001
EMBED = 32
HIDDEN = 64
BATCH = 16384

HB = BATCH // 2
C0 = 49920
C1 = 50048
CMAIN = C0 + C1
NTAIL = VOCAB - CMAIN


def _sc_gather_sum(tables_t, tails_t, ids_1d):
    mesh = plsc.VectorSubcoreMesh(core_axis_name="c", subcore_axis_name="s")

    @functools.partial(
        pl.kernel,
        mesh=mesh,
        out_type=jax.ShapeDtypeStruct((EMBED * BATCH,), jnp.float32),
        compiler_params=pltpu.CompilerParams(needs_layout_passes=False),
        scratch_types=[
            pltpu.VMEM((C1,), jnp.float32),
            pltpu.VMEM((8, NTAIL), jnp.float32),
            pltpu.VMEM((HB,), jnp.int32),
            pltpu.VMEM((HB,), jnp.float32),
            pltpu.VMEM_SHARED((8, CMAIN), jnp.float32),
            pltpu.SemaphoreType.DMA,
        ],
    )
    def k(tab_hbm, tails_hbm, ids_hbm, out_hbm,
          row_v, tail_v, ids_v, acc_v, slab_s, sem):
        c = lax.axis_index("c")
        s = lax.axis_index("s")
        er = s % 8
        half = s // 8

        steps = [(g, f) for g in range(2) for f in range(N_CAT)]

        def stage(g, f, async_=False):
            eoff = pl.multiple_of(c * 16 + g * 8, 8)
            for st, (off, ln) in ((0, (0, C0)), (8, (C0, C1))):
                @pl.when(s == st)
                def _(off=off, ln=ln):
                    cp = pltpu.make_async_copy(
                        tab_hbm.at[f, pl.ds(eoff, 8), pl.ds(off, ln)],
                        slab_s.at[:, pl.ds(off, ln)], sem)
                    cp.start()
                    if not async_:
                        cp.wait()

        def stage_wait(g, f):
            eoff = pl.multiple_of(c * 16 + g * 8, 8)
            for st, (off, ln) in ((0, (0, C0)), (8, (C0, C1))):
                @pl.when(s == st)
                def _(off=off, ln=ln):
                    pltpu.make_async_copy(
                        tab_hbm.at[f, pl.ds(eoff, 8), pl.ds(off, ln)],
                        slab_s.at[:, pl.ds(off, ln)], sem).wait()

        def pass0(first):
            def inner(i, carry):
                o = i * 16
                idxv = ids_v[pl.ds(o, 16)]
                m = idxv < C0
                lidc = lax.min(idxv, C0 - 1)
                vals = jnp.where(m, plsc.load_gather(row_v, [lidc]), 0.0)
                if first:
                    acc_v[pl.ds(o, 16)] = vals
                else:
                    plsc.addupdate(acc_v.at[pl.ds(o, 16)], vals)
                return carry

            lax.fori_loop(0, HB // 16, inner, 0)

        def pass1(er_vec):
            def inner(i, carry):
                o = i * 16
                idxv = ids_v[pl.ds(o, 16)]
                lid = idxv - C0
                m = (lid >= 0) & (idxv < CMAIN)
                lidc = lax.max(lax.min(lid, C1 - 1), 0)
                vals = jnp.where(m, plsc.load_gather(row_v, [lidc]), 0.0)
                tm = idxv >= CMAIN
                tl = lax.max(idxv - CMAIN, 0)
                tvals = jnp.where(
                    tm, plsc.load_gather(tail_v, [er_vec, tl]), 0.0)
                plsc.addupdate(acc_v.at[pl.ds(o, 16)], vals + tvals)
                return carry

            lax.fori_loop(0, HB // 16, inner, 0)

        stage(0, 0, async_=False)
        plsc.subcore_barrier()

        er_vec = jnp.full((16,), 0, dtype=jnp.int32) + er

        for si, (g, f) in enumerate(steps):
            eoff = pl.multiple_of(c * 16 + g * 8, 8)
            pltpu.sync_copy(
                ids_hbm.at[pl.ds(f * BATCH + half * HB, HB)], ids_v)
            pltpu.sync_copy(tails_hbm.at[f, pl.ds(eoff, 8)], tail_v)
            pltpu.sync_copy(slab_s.at[er, pl.ds(0, C0)],
                            row_v.at[pl.ds(0, C0)])
            pass0(f == 0)
            pltpu.sync_copy(slab_s.at[er, pl.ds(C0, C1)], row_v)
            plsc.subcore_barrier()
            nxt = steps[si + 1] if si + 1 < len(steps) else None
            if nxt is not None:
                stage(*nxt, async_=True)
            pass1(er_vec)
            if f == N_CAT - 1:
                e = c * 16 + g * 8 + er
                pltpu.sync_copy(
                    acc_v, out_hbm.at[pl.ds(e * BATCH + half * HB, HB)])
            if nxt is not None:
                stage_wait(*nxt)
            plsc.subcore_barrier()

    return k(tables_t, tails_t, ids_1d)


def _dense_tail(obj_t, nums, W1, W2):
    BM = 2048

    def body(obj_ref, num_ref, w1_ref, w2_ref, out_ref):
        n1 = lax.dot_general(
            num_ref[...], w1_ref[...], (((1,), (1,)), ((), ())),
            preferred_element_type=jnp.float32)
        a = lax.dot_general(
            obj_ref[...], w2_ref[...], (((0,), (1,)), ((), ())),
            preferred_element_type=jnp.float32)
        out_ref[...] = a + lax.dot_general(
            n1, w2_ref[...], (((1,), (1,)), ((), ())),
            preferred_element_type=jnp.float32)

    return pl.pallas_call(
        body,
        grid=(BATCH // BM,),
        in_specs=[
            pl.BlockSpec((EMBED, BM), lambda i: (0, i)),
            pl.BlockSpec((BM, N_NUM), lambda i: (i, 0)),
            pl.BlockSpec((EMBED, N_NUM), lambda i: (0, 0)),
            pl.BlockSpec((HIDDEN, EMBED), lambda i: (0, 0)),
        ],
        out_specs=pl.BlockSpec((BM, HIDDEN), lambda i: (i, 0)),
        out_shape=jax.ShapeDtypeStruct((BATCH, HIDDEN), jnp.float32),
    )(obj_t, nums, W1, W2)


def kernel(edge_feats, tables, W1, W2):
    ids_1d = edge_feats[:, :N_CAT].astype(jnp.int32).T.reshape(-1)
    tables_t = tables.transpose(0, 2, 1)
    tails_t = lax.slice(tables_t, (0, 0, CMAIN), (N_CAT, EMBED, VOCAB))
    obj_flat = _sc_gather_sum(tables_t, tails_t, ids_1d)
    obj_t = obj_flat.reshape(EMBED, BATCH)
    return _dense_tail(obj_t, edge_feats[:, N_CAT:], W1, W2)


# speedup vs baseline: 21.8793x; 1.2991x over previous
import functools

import jax
import jax.numpy as jnp
from jax import lax
from jax.experimental import pallas as pl
from jax.experimental.pallas import tpu as pltpu
from jax.experimental.pallas import tpu_sc as plsc

N_CAT = 26
N_NUM = 13
VOCAB = 100001
EMBED = 32
HIDDEN = 64
BATCH = 16384

HB = BATCH // 2
C0 = 49920
C1 = 50048
CMAIN = C0 + C1
NTAIL = VOCAB - CMAIN


def _sc_gather_sum(tables_t, tails_t, ids_1d):
    mesh = plsc.VectorSubcoreMesh(core_axis_name="c", subcore_axis_name="s")

    @functools.partial(
        pl.kernel,
        mesh=mesh,
        out_type=jax.ShapeDtypeStruct((EMBED * BATCH,), jnp.float32),
        compiler_params=pltpu.CompilerParams(needs_layout_passes=False),
        scratch_types=[
            pltpu.VMEM((C1,), jnp.float32),
            pltpu.VMEM((2, 8, NTAIL), jnp.float32),
            pltpu.VMEM((2, HB), jnp.int32),
            pltpu.VMEM((HB,), jnp.float32),
            pltpu.VMEM_SHARED((8, CMAIN), jnp.float32),
            pltpu.SemaphoreType.DMA,
            pltpu.SemaphoreType.DMA,
        ],
    )
    def k(tab_hbm, tails_hbm, ids_hbm, out_hbm,
          row_v, tail_v, ids_v, acc_v, slab_s, sem, sem2):
        c = lax.axis_index("c")
        s = lax.axis_index("s")
        er = s % 8
        half = s // 8

        def stage(g, f, async_=False):
            eoff = pl.multiple_of(c * 16 + g * 8, 8)
            for st, (off, ln) in ((0, (0, C0)), (8, (C0, C1))):
                @pl.when(s == st)
                def _(off=off, ln=ln):
                    cp = pltpu.make_async_copy(
                        tab_hbm.at[f, pl.ds(eoff, 8), pl.ds(off, ln)],
                        slab_s.at[:, pl.ds(off, ln)], sem)
                    cp.start()
                    if not async_:
                        cp.wait()

        def stage_wait(g, f):
            eoff = pl.multiple_of(c * 16 + g * 8, 8)
            for st, (off, ln) in ((0, (0, C0)), (8, (C0, C1))):
                @pl.when(s == st)
                def _(off=off, ln=ln):
                    pltpu.make_async_copy(
                        tab_hbm.at[f, pl.ds(eoff, 8), pl.ds(off, ln)],
                        slab_s.at[:, pl.ds(off, ln)], sem).wait()

        def ids_copy(f, b):
            return pltpu.make_async_copy(
                ids_hbm.at[pl.ds(
                    pl.multiple_of(f * BATCH + half * HB, 8), HB)],
                ids_v.at[b], sem2)

        def tail_copy(f, b, eoff):
            return pltpu.make_async_copy(
                tails_hbm.at[f, pl.ds(eoff, 8)], tail_v.at[b], sem2)

        def pass0(b):
            @plsc.parallel_loop(0, HB, step=16, unroll=8)
            def _(o):
                idxv = ids_v[b, pl.ds(o, 16)]
                m = idxv < C0
                lidc = lax.min(idxv, C0 - 1)
                vals = jnp.where(m, plsc.load_gather(row_v, [lidc]), 0.0)
                plsc.addupdate(acc_v.at[pl.ds(o, 16)], vals)

        def pass1(b, b_vec, er_vec):
            @plsc.parallel_loop(0, HB, step=16, unroll=8)
            def _(o):
                idxv = ids_v[b, pl.ds(o, 16)]
                lid = idxv - C0
                m = (lid >= 0) & (idxv < CMAIN)
                lidc = lax.max(lax.min(lid, C1 - 1), 0)
                vals = jnp.where(m, plsc.load_gather(row_v, [lidc]), 0.0)
                tm = idxv >= CMAIN
                tl = lax.max(idxv - CMAIN, 0)
                tvals = jnp.where(
                    tm, plsc.load_gather(tail_v, [b_vec, er_vec, tl]), 0.0)
                plsc.addupdate(acc_v.at[pl.ds(o, 16)], vals + tvals)

        er_vec = jnp.full((16,), 0, dtype=jnp.int32) + er

        stage(0, 0, async_=False)
        plsc.subcore_barrier()

        for g in range(2):
            @plsc.parallel_loop(0, HB, step=16, unroll=8)
            def _(o):
                acc_v[pl.ds(o, 16)] = jnp.zeros((16,), jnp.float32)

            eoff0 = pl.multiple_of(c * 16 + g * 8, 8)
            ids_copy(0, 0).start()
            tail_copy(0, 0, eoff0).start()
            ids_copy(0, 0).wait()
            tail_copy(0, 0, eoff0).wait()

            def step(f, carry, g=g):
                eoff = pl.multiple_of(c * 16 + g * 8, 8)
                b = lax.rem(f, 2)
                b_vec = jnp.full((16,), 0, dtype=jnp.int32) + b

                @pl.when(f > 0)
                def _():
                    ids_copy(f, b).wait()
                    tail_copy(f, b, eoff).wait()

                @pl.when(f < N_CAT - 1)
                def _():
                    ids_copy(f + 1, 1 - b).start()
                    tail_copy(f + 1, 1 - b, eoff).start()

                pltpu.sync_copy(slab_s.at[er, pl.ds(0, C0)],
                                row_v.at[pl.ds(0, C0)])
                pass0(b)
                pltpu.sync_copy(slab_s.at[er, pl.ds(C0, C1)], row_v)
                plsc.subcore_barrier()

                @pl.when(f < N_CAT - 1)
                def _():
                    stage(g, f + 1, async_=True)
                pass1(b, b_vec, er_vec)

                @pl.when(f < N_CAT - 1)
                def _():
                    stage_wait(g, f + 1)
                plsc.subcore_barrier()
                return carry

            lax.fori_loop(0, N_CAT, step, 0)

            e = c * 16 + g * 8 + er
            pltpu.sync_copy(
                acc_v, out_hbm.at[pl.ds(e * BATCH + half * HB, HB)])
            if g == 0:
                stage(1, 0, async_=False)
                plsc.subcore_barrier()

    return k(tables_t, tails_t, ids_1d)


def _dense_tail(obj_t, nums, W1, W2):
    BM = 2048

    def body(obj_ref, num_ref, w1_ref, w2_ref, out_ref):
        n1 = lax.dot_general(
            num_ref[...], w1_ref[...], (((1,), (1,)), ((), ())),
            preferred_element_type=jnp.float32)
        a = lax.dot_general(
            obj_ref[...], w2_ref[...], (((0,), (1,)), ((), ())),
            preferred_element_type=jnp.float32)
        out_ref[...] = a + lax.dot_general(
            n1, w2_ref[...], (((1,), (1,)), ((), ())),
            preferred_element_type=jnp.float32)

    return pl.pallas_call(
        body,
        grid=(BATCH // BM,),
        in_specs=[
            pl.BlockSpec((EMBED, BM), lambda i: (0, i)),
            pl.BlockSpec((BM, N_NUM), lambda i: (i, 0)),
            pl.BlockSpec((EMBED, N_NUM), lambda i: (0, 0)),
            pl.BlockSpec((HIDDEN, EMBED), lambda i: (0, 0)),
        ],
        out_specs=pl.BlockSpec((BM, HIDDEN), lambda i: (i, 0)),
        out_shape=jax.ShapeDtypeStruct((BATCH, HIDDEN), jnp.float32),
    )(obj_t, nums, W1, W2)


def kernel(edge_feats, tables, W1, W2):
    ids_1d = edge_feats[:, :N_CAT].astype(jnp.int32).T.reshape(-1)
    tables_t = tables.transpose(0, 2, 1)
    tails_t = lax.slice(tables_t, (0, 0, CMAIN), (N_CAT, EMBED, VOCAB))
    obj_flat = _sc_gather_sum(tables_t, tails_t, ids_1d)
    obj_t = obj_flat.reshape(EMBED, BATCH)
    return _dense_tail(obj_t, edge_feats[:, N_CAT:], W1, W2)


# speedup vs baseline: 25.9203x; 1.1847x over previous
import functools

import jax
import jax.numpy as jnp
from jax import lax
from jax.experimental import pallas as pl
from jax.experimental.pallas import tpu as pltpu
from jax.experimental.pallas import tpu_sc as plsc

N_CAT = 26
N_NUM = 13
VOCAB = 100001
EMBED = 32
HIDDEN = 64
BATCH = 16384

HB = BATCH // 2
C0 = 49920
C1 = 50048
CMAIN = C0 + C1
NTAIL = VOCAB - CMAIN


def _sc_gather_sum(tables_t, tails_t, ids_1d):
    mesh = plsc.VectorSubcoreMesh(core_axis_name="c", subcore_axis_name="s")

    @functools.partial(
        pl.kernel,
        mesh=mesh,
        out_type=jax.ShapeDtypeStruct((EMBED * BATCH,), jnp.float32),
        compiler_params=pltpu.CompilerParams(needs_layout_passes=False),
        scratch_types=[
            pltpu.VMEM((C1,), jnp.float32),
            pltpu.VMEM((2, 8, NTAIL), jnp.float32),
            pltpu.VMEM((2, HB), jnp.int32),
            pltpu.VMEM((HB,), jnp.float32),
            pltpu.VMEM_SHARED((8, C0), jnp.float32),
            pltpu.VMEM_SHARED((8, C1), jnp.float32),
            pltpu.SemaphoreType.DMA,
            pltpu.SemaphoreType.DMA,
        ],
    )
    def k(tab_hbm, tails_hbm, ids_hbm, out_hbm,
          row_v, tail_v, ids_v, acc_v, slab_a, slab_b, sem, sem2):
        c = lax.axis_index("c")
        s = lax.axis_index("s")
        er = s % 8
        half = s // 8

        _AV = ((0, 0, 24960), (4, 24960, 24960))
        _BV = ((8, 0, 24960), (12, 24960, 25088))

        def _stage_half(g, f, variants, base, dst, async_=False, wait=False):
            eoff = pl.multiple_of(c * 16 + g * 8, 8)
            for st, off, ln in variants:
                @pl.when(s == st)
                def _(off=off, ln=ln):
                    cp = pltpu.make_async_copy(
                        tab_hbm.at[f, pl.ds(eoff, 8),
                                   pl.ds(base + off, ln)],
                        dst.at[:, pl.ds(off, ln)], sem)
                    if wait:
                        cp.wait()
                    else:
                        cp.start()
                        if not async_:
                            cp.wait()

        def stage_a(g, f, **kw):
            _stage_half(g, f, _AV, 0, slab_a, **kw)

        def stage_b(g, f, **kw):
            _stage_half(g, f, _BV, C0, slab_b, **kw)

        def ids_copy(f, b):
            return pltpu.make_async_copy(
                ids_hbm.at[pl.ds(
                    pl.multiple_of(f * BATCH + half * HB, 8), HB)],
                ids_v.at[b], sem2)

        def tail_copy(f, b, eoff):
            return pltpu.make_async_copy(
                tails_hbm.at[f, pl.ds(eoff, 8)], tail_v.at[b], sem2)

        def pass0(b):
            @plsc.parallel_loop(0, HB, step=16, unroll=8)
            def _(o):
                idxv = ids_v[b, pl.ds(o, 16)]
                m = idxv < C0
                lidc = lax.min(idxv, C0 - 1)
                vals = jnp.where(m, plsc.load_gather(row_v, [lidc]), 0.0)
                plsc.addupdate(acc_v.at[pl.ds(o, 16)], vals)

        def pass1(b, b_vec, er_vec):
            @plsc.parallel_loop(0, HB, step=16, unroll=8)
            def _(o):
                idxv = ids_v[b, pl.ds(o, 16)]
                lid = idxv - C0
                m = (lid >= 0) & (idxv < CMAIN)
                lidc = lax.max(lax.min(lid, C1 - 1), 0)
                vals = jnp.where(m, plsc.load_gather(row_v, [lidc]), 0.0)
                tm = idxv >= CMAIN
                tl = lax.max(idxv - CMAIN, 0)
                tvals = jnp.where(
                    tm, plsc.load_gather(tail_v, [b_vec, er_vec, tl]), 0.0)
                plsc.addupdate(acc_v.at[pl.ds(o, 16)], vals + tvals)

        er_vec = jnp.full((16,), 0, dtype=jnp.int32) + er

        stage_a(0, 0, async_=False)
        stage_b(0, 0, async_=False)
        plsc.subcore_barrier()

        for g in range(2):
            @plsc.parallel_loop(0, HB, step=16, unroll=8)
            def _(o):
                acc_v[pl.ds(o, 16)] = jnp.zeros((16,), jnp.float32)

            eoff0 = pl.multiple_of(c * 16 + g * 8, 8)
            ids_copy(0, 0).start()
            tail_copy(0, 0, eoff0).start()
            ids_copy(0, 0).wait()
            tail_copy(0, 0, eoff0).wait()

            def step(f, carry, g=g):
                eoff = pl.multiple_of(c * 16 + g * 8, 8)
                b = lax.rem(f, 2)
                b_vec = jnp.full((16,), 0, dtype=jnp.int32) + b

                @pl.when(f > 0)
                def _():
                    ids_copy(f, b).wait()
                    tail_copy(f, b, eoff).wait()

                @pl.when(f < N_CAT - 1)
                def _():
                    ids_copy(f + 1, 1 - b).start()
                    tail_copy(f + 1, 1 - b, eoff).start()

                pltpu.sync_copy(slab_a.at[er], row_v.at[pl.ds(0, C0)])
                plsc.subcore_barrier()
                @pl.when(f < N_CAT - 1)
                def _():
                    stage_a(g, f + 1, async_=True)
                pass0(b)
                pltpu.sync_copy(slab_b.at[er], row_v)
                plsc.subcore_barrier()
                @pl.when(f < N_CAT - 1)
                def _():
                    stage_b(g, f + 1, async_=True)
                pass1(b, b_vec, er_vec)

                @pl.when(f < N_CAT - 1)
                def _():
                    stage_a(g, f + 1, wait=True)
                    stage_b(g, f + 1, wait=True)
                plsc.subcore_barrier()
                return carry

            lax.fori_loop(0, N_CAT, step, 0)

            e = c * 16 + g * 8 + er
            pltpu.sync_copy(
                acc_v, out_hbm.at[pl.ds(e * BATCH + half * HB, HB)])
            if g == 0:
                stage_a(1, 0, async_=False)
                stage_b(1, 0, async_=False)
                plsc.subcore_barrier()

    return k(tables_t, tails_t, ids_1d)


def _dense_tail(obj_t, nums, W1, W2):
    BM = 2048

    def body(obj_ref, num_ref, w1_ref, w2_ref, out_ref):
        n1 = lax.dot_general(
            num_ref[...], w1_ref[...], (((1,), (1,)), ((), ())),
            preferred_element_type=jnp.float32)
        a = lax.dot_general(
            obj_ref[...], w2_ref[...], (((0,), (1,)), ((), ())),
            preferred_element_type=jnp.float32)
        out_ref[...] = a + lax.dot_general(
            n1, w2_ref[...], (((1,), (1,)), ((), ())),
            preferred_element_type=jnp.float32)

    return pl.pallas_call(
        body,
        grid=(BATCH // BM,),
        in_specs=[
            pl.BlockSpec((EMBED, BM), lambda i: (0, i)),
            pl.BlockSpec((BM, N_NUM), lambda i: (i, 0)),
            pl.BlockSpec((EMBED, N_NUM), lambda i: (0, 0)),
            pl.BlockSpec((HIDDEN, EMBED), lambda i: (0, 0)),
        ],
        out_specs=pl.BlockSpec((BM, HIDDEN), lambda i: (i, 0)),
        out_shape=jax.ShapeDtypeStruct((BATCH, HIDDEN), jnp.float32),
    )(obj_t, nums, W1, W2)


def kernel(edge_feats, tables, W1, W2):
    ids_1d = edge_feats[:, :N_CAT].astype(jnp.int32).T.reshape(-1)
    tables_t = tables.transpose(0, 2, 1)
    tails_t = lax.slice(tables_t, (0, 0, CMAIN), (N_CAT, EMBED, VOCAB))
    obj_flat = _sc_gather_sum(tables_t, tails_t, ids_1d)
    obj_t = obj_flat.reshape(EMBED, BATCH)
    return _dense_tail(obj_t, edge_feats[:, N_CAT:], W1, W2)
